# baseline (speedup 1.0000x reference)
import functools

import jax
import jax.numpy as jnp
from jax import lax
from jax.experimental import pallas as pl
from jax.experimental.pallas import tpu as pltpu
from jax.experimental.pallas import tpu_sc as plsc

_B, _C = 16, 384
_T = 24 * 24
_K, _D = 1024, 32
_N = _B * _T
_EPS = 1e-5

_BB = 2
_T2 = _BB * _T

_NC, _NS = 2, 16
_NW = _NC * _NS
_BPW = _N // _NW
_CH = 4
_RPC = _BPW // _CH


def _table_body(emb_ref, embt_ref, w1_ref, b1_ref, w2_ref, b2_ref, g_ref,
                bt_ref, out_ref, e2_ref):
    bf = lambda a: a.astype(jnp.bfloat16)
    h = jnp.maximum(
        jnp.dot(bf(emb_ref[...]), bf(w1_ref[...]),
                preferred_element_type=jnp.float32) + b1_ref[...], 0.0)
    q = jnp.dot(bf(h), bf(w2_ref[...]),
                preferred_element_type=jnp.float32) + b2_ref[...]
    m = jnp.mean(q, axis=-1, keepdims=True)
    v = jnp.mean((q - m) ** 2, axis=-1, keepdims=True)
    out_ref[...] = (q - m) / jnp.sqrt(v + _EPS) * g_ref[...] + bt_ref[...]
    embt = embt_ref[...]
    e2_ref[...] = jnp.sum(embt * embt, axis=0, keepdims=True)


def _table(emb, embt, w1, b1, w2, b2, g, bt):
    return pl.pallas_call(
        _table_body,
        out_shape=[jax.ShapeDtypeStruct((_K, _C), jnp.float32),
                   jax.ShapeDtypeStruct((1, _K), jnp.float32)],
    )(emb, embt, w1, b1, w2, b2, g, bt)


def _front_body(f_ref, w1_ref, b1_ref, w2_ref, b2_ref, g_ref, bt_ref,
                embt_ref, e2_ref, idx_ref):
    x = jnp.concatenate([f_ref[0].T, f_ref[1].T], axis=0)
    bf = lambda a: a.astype(jnp.bfloat16)
    h = jnp.maximum(
        jnp.dot(bf(x), bf(w1_ref[...]),
                preferred_element_type=jnp.float32) + b1_ref[...], 0.0)
    z = jnp.dot(bf(h), bf(w2_ref[...]),
                preferred_element_type=jnp.float32) + b2_ref[...]
    m = jnp.mean(z, axis=-1, keepdims=True)
    v = jnp.mean((z - m) ** 2, axis=-1, keepdims=True)
    zn = (z - m) / jnp.sqrt(v + _EPS) * g_ref[...] + bt_ref[...]
    scores = e2_ref[...] - 2.0 * jnp.dot(zn, embt_ref[...],
                                         preferred_element_type=jnp.float32)
    iota = lax.broadcasted_iota(jnp.int32, (_T2, _K), 1)
    mn = jnp.min(scores, axis=1, keepdims=True)
    idx = jnp.min(jnp.where(scores == mn, iota, _K), axis=1)
    idx_ref[0, 0, :] = idx


def _front(f3, w1, b1, w2, b2, g, bt, embt, e2):
    full = lambda *s: pl.BlockSpec(s, lambda i: (0,) * len(s))
    return pl.pallas_call(
        _front_body,
        grid=(_B // _BB,),
        in_specs=[
            pl.BlockSpec((_BB, _C, _T), lambda i: (i, 0, 0)),
            full(_C, _C), full(1, _C), full(_C, _D), full(1, _D),
            full(1, _D), full(1, _D), full(_D, _K), full(1, _K),
        ],
        out_specs=pl.BlockSpec((1, 1, _T2), lambda i: (i, 0, 0)),
        out_shape=jax.ShapeDtypeStruct((_B // _BB, 1, _T2), jnp.int32),
    )(f3, w1, b1, w2, b2, g, bt, embt, e2)


def _onehot_body(idx_ref, enc_ref):
    iota = lax.broadcasted_iota(jnp.int32, (_T2, _K), 1)
    enc_ref[0] = (iota == idx_ref[0, 0, :][:, None]).astype(jnp.float32)


def _onehot(idx3):
    return pl.pallas_call(
        _onehot_body,
        grid=(_B // _BB,),
        in_specs=[pl.BlockSpec((1, 1, _T2), lambda i: (i, 0, 0))],
        out_specs=pl.BlockSpec((1, _T2, _K), lambda i: (i, 0, 0)),
        out_shape=jax.ShapeDtypeStruct((_B // _BB, _T2, _K), jnp.float32),
    )(idx3)


def _gather_body(table_hbm, idx_hbm, out_hbm, idx_v, b0, b1, b2, b3,
                 g0, g1, g2, g3, w0, w1, w2, w3):
    wid = lax.axis_index("s") * _NC + lax.axis_index("c")
    base = wid * _BPW
    pltpu.sync_copy(idx_hbm.at[pl.ds(base, _BPW)], idx_v)
    bufs = (b0, b1, b2, b3)
    gsems = (g0, g1, g2, g3)
    wsems = (w0, w1, w2, w3)
    gets = [
        pltpu.async_copy(table_hbm.at[idx_v.at[pl.ds(c * _RPC, _RPC)]],
                         bufs[c], gsems[c])
        for c in range(_CH)
    ]
    puts = []
    for c in range(_CH):
        gets[c].wait()
        puts.append(pltpu.async_copy(
            bufs[c], out_hbm.at[pl.ds(base + c * _RPC, _RPC)], wsems[c]))
    for p in puts:
        p.wait()


def _gather(table, idx):
    mesh = plsc.VectorSubcoreMesh(core_axis_name="c", subcore_axis_name="s")
    kern = functools.partial(
        pl.kernel, mesh=mesh,
        out_type=jax.ShapeDtypeStruct((_N, _C), jnp.float32),
        scratch_types=(
            [pltpu.VMEM((_BPW,), jnp.int32)]
            + [pltpu.VMEM((_RPC, _C), jnp.float32) for _ in range(_CH)]
            + [pltpu.SemaphoreType.DMA for _ in range(2 * _CH)]
        ),
    )(_gather_body)
    return kern(table, idx)


def kernel(features, y, W_in1, b_in1, W_in2, b_in2, g_in, beta_in, emb,
           W_out1, b_out1, W_out2, b_out2, g_out, beta_out):
    f3 = features.reshape(_B, _C, _T)
    row = lambda a: a.reshape(1, -1)
    embt = emb.T
    table, e2 = _table(emb, embt, W_out1, row(b_out1), W_out2, row(b_out2),
                       row(g_out), row(beta_out))
    idx3 = _front(f3, W_in1, row(b_in1), W_in2, row(b_in2),
                  row(g_in), row(beta_in), embt, e2)
    idx = idx3.reshape(_N)
    q = _gather(table, idx)
    enc = _onehot(idx3)
    quantized = q.reshape(_B, _T, _C)
    return (quantized, idx.reshape(_N, 1), enc.reshape(_B, _T, _K))
